# baseline (speedup 1.0000x reference)
import functools

import jax
import jax.numpy as jnp
from jax import lax
from jax.experimental import pallas as pl
from jax.experimental.pallas import tpu as pltpu
from jax.experimental.pallas import tpu_sc as plsc

L, B, NP, NG, S, P, C = 6, 4, 1000, 50, 20, 20, 3
D = P * 2
ALPHA, GAMMA = 0.25, 2.0
W_CLS, W_PTS, W_DIR = 2.0, 5.0, 0.005
COST_CLS, COST_PTS = 2.0, 5.0
EPS = 1e-8

NTC = 768
NSC = NP - NTC
NSCP = 256
DP = 48
PROBS = L * B


def _cls_rows(clsT_r):
    x = clsT_r[0, 0]
    p = jax.nn.sigmoid(x)
    one_m_p = 1.0 - p
    pos = ALPHA * (one_m_p * one_m_p) * (-jnp.log(p + EPS))
    neg = (1.0 - ALPHA) * (p * p) * (-jnp.log(one_m_p + EPS))
    return pos - neg, neg


def _cls_T(pn, oh):
    return (oh[:, 0:1] * pn[0:1, :]
            + oh[:, 1:2] * pn[1:2, :]
            + oh[:, 2:3] * pn[2:3, :])


def _rowmin_arg_pts(cost, rm, n):
    rowmin = jnp.min(cost, axis=1, keepdims=True)
    l_iota = jax.lax.broadcasted_iota(jnp.int32, cost.shape, 1)
    argidx = jnp.min(jnp.where(cost == rowmin, l_iota, n),
                     axis=1, keepdims=True)
    ptsv = jnp.sum(jnp.where(l_iota == argidx, rm, 0.0),
                   axis=1, keepdims=True)
    return rowmin, argidx, ptsv


def _tc_cost_kernel(predT_r, gt3_r, clsT_r, oh_r, out_r, cand_r, rm_ref):
    pred = predT_r[0, 0]
    big = jnp.float32(3.4e38)
    LC = 256

    def s_body(s, _):
        gts = gt3_r[0, s]
        for lo in range(0, NTC, LC):
            hi = lo + LC
            acc = jnp.abs(gts[:, 0:1] - pred[0:1, lo:hi])
            for d in range(1, D):
                acc = acc + jnp.abs(gts[:, d:d + 1] - pred[d:d + 1, lo:hi])
            rm_ref[:, lo:hi] = jnp.minimum(rm_ref[:, lo:hi], acc)
        return 0

    rm_ref[...] = jnp.full((NG, NTC), big, jnp.float32)
    jax.lax.fori_loop(0, S, s_body, 0)
    runmin = rm_ref[...]

    pn, neg = _cls_rows(clsT_r)
    cls_T = _cls_T(pn, oh_r[0])
    cost = cls_T[:, :NTC] * COST_CLS + runmin * COST_PTS
    rowmin, argidx, ptsv = _rowmin_arg_pts(cost, runmin, NTC)

    bg = jnp.sum(neg)
    riota = jax.lax.broadcasted_iota(jnp.int32, (8, 128), 0)
    out_r[0, 0] = jnp.where(riota == 3, bg, 0.0)

    ciota = jax.lax.broadcasted_iota(jnp.int32, (NG, 8), 1)
    cand_r[0, 0] = (jnp.where(ciota == 0, rowmin, 0.0)
                    + jnp.where(ciota == 1, argidx.astype(jnp.float32), 0.0)
                    + jnp.where(ciota == 2, ptsv, 0.0))


NGC = 10
NCHUNK = NG // NGC


def _sc_runmin_body(pred_hbm, gt_hbm, out_hbm, gtbuf, predv, outv):
    c = lax.axis_index("c")
    sid = lax.axis_index("s")

    def k_body(k, _):
        p = 2 * k + c
        b = p % B
        pltpu.sync_copy(pred_hbm.at[p, sid], predv)

        def cg_body(cg, _):
            pltpu.sync_copy(gt_hbm.at[b, pl.ds(cg * NGC * S, NGC * S)],
                            gtbuf)

            def ng_body(ngl, _):
                def s_body(s, rm):
                    j = ngl * S + s
                    g0 = gtbuf[j, pl.ds(0, 16)]
                    g1 = gtbuf[j, pl.ds(16, 16)]
                    g2 = gtbuf[j, pl.ds(32, 16)]
                    gc = (g0, g1, g2)
                    acc = jnp.abs(predv[0, :] - g0[0])
                    for d in range(1, D):
                        acc = acc + jnp.abs(predv[d, :] - gc[d // 16][d % 16])
                    return jnp.minimum(rm, acc)

                rm = jax.lax.fori_loop(
                    0, S, s_body, jnp.full((16,), 3.4e38, jnp.float32))
                outv[cg * NGC + ngl, :] = rm
                return 0

            jax.lax.fori_loop(0, NGC, ng_body, 0)
            return 0

        jax.lax.fori_loop(0, NCHUNK, cg_body, 0)
        pltpu.sync_copy(outv, out_hbm.at[p, sid])
        return 0

    jax.lax.fori_loop(0, PROBS // 2, k_body, 0)


def _merge_kernel(scrm_r, cand_r, predT_r, gt3_r, gtflat_r, clsT_r, oh_r,
                  out_r):
    big = jnp.float32(3.4e38)
    pred = predT_r[0, 0]
    pn, _ = _cls_rows(clsT_r)
    cls_T = _cls_T(pn, oh_r[0])

    sc_rm = scrm_r[0, 0][:, :NSC]
    sc_cost = cls_T[:, NTC:] * COST_CLS + sc_rm * COST_PTS
    sc_min, sc_idx, sc_pts = _rowmin_arg_pts(sc_cost, sc_rm, NSC)

    cand = cand_r[0, 0]
    tc_min = cand[:, 0:1]
    tc_idx = cand[:, 1:2].astype(jnp.int32)
    tc_pts = cand[:, 2:3]

    take_sc = sc_min < tc_min
    m_idx = jnp.where(take_sc, sc_idx + NTC, tc_idx)
    m_pts = jnp.where(take_sc, sc_pts, tc_pts)

    l_iota = jax.lax.broadcasted_iota(jnp.int32, (NG, NP), 1)
    onehot_assign = l_iota == m_idx

    ng_iota = jax.lax.broadcasted_iota(jnp.int32, (NG, NP), 0)
    colmax = jnp.max(jnp.where(onehot_assign, ng_iota, -1),
                     axis=0, keepdims=True)
    winner = onehot_assign & (ng_iota == colmax)
    wf = winner.astype(jnp.float32)

    roww = jnp.max(wf, axis=1, keepdims=True)
    num_pos = jnp.sum(roww)
    sum_pts = jnp.sum(roww * m_pts)
    sum_corr = jnp.sum(wf * cls_T)

    gpred = jax.lax.dot_general(
        wf, pred, (((1,), (1,)), ((), ())),
        preferred_element_type=jnp.float32)

    srunmin = jnp.full((NG, 1), big, jnp.float32)
    srunarg = jnp.zeros((NG, 1), jnp.int32)
    for s in range(S):
        g = gt3_r[0, s]
        dsum = jnp.sum(jnp.abs(gpred - g), axis=1, keepdims=True)
        better = dsum < srunmin
        srunarg = jnp.where(better, s, srunarg)
        srunmin = jnp.minimum(srunmin, dsum)

    ng_col = jax.lax.broadcasted_iota(jnp.int32, (NG, 1), 0)
    jcol = ng_col * S + srunarg
    j_iota = jax.lax.broadcasted_iota(jnp.int32, (NG, NG * S), 1)
    ohns = (j_iota == jcol).astype(jnp.float32)
    gtd = jax.lax.dot_general(
        ohns, gtflat_r[0], (((1,), (0,)), ((), ())),
        preferred_element_type=jnp.float32)

    pd = gpred[:, 2:] - gpred[:, :-2]
    td = gtd[:, 2:] - gtd[:, :-2]
    r2 = jax.lax.broadcasted_iota(jnp.int32, (D - 2, P - 1), 0) // 2
    c2 = jax.lax.broadcasted_iota(jnp.int32, (D - 2, P - 1), 1)
    pair_m = (r2 == c2).astype(jnp.float32)
    stk = jnp.concatenate([pd * td, pd * pd, td * td], axis=0)
    res = jax.lax.dot_general(stk, pair_m, (((1,), (0,)), ((), ())),
                              preferred_element_type=jnp.float32)
    dots = res[0:NG]
    pdn = res[NG:2 * NG]
    tdn = res[2 * NG:3 * NG]
    cos = dots / (jnp.sqrt(pdn) * jnp.sqrt(tdn) + EPS)
    sum_dir = jnp.sum((1.0 - cos) * roww)

    riota = jax.lax.broadcasted_iota(jnp.int32, (8, 128), 0)
    out_r[0, 0] = (jnp.where(riota == 0, num_pos, 0.0)
                   + jnp.where(riota == 1, sum_pts, 0.0)
                   + jnp.where(riota == 2, sum_corr, 0.0)
                   + jnp.where(riota == 4, sum_dir, 0.0))


def _sc_call(pred_sc, gtflat):
    mesh = plsc.VectorSubcoreMesh(core_axis_name="c", subcore_axis_name="s")
    fn = functools.partial(
        pl.kernel,
        mesh=mesh,
        out_type=jax.ShapeDtypeStruct((PROBS, 16, NG, 16), jnp.float32),
        scratch_types=[
            pltpu.VMEM((NGC * S, DP), jnp.float32),
            pltpu.VMEM((D, 16), jnp.float32),
            pltpu.VMEM((NG, 16), jnp.float32),
        ],
    )(_sc_runmin_body)
    return fn(pred_sc, gtflat)


@jax.jit
def kernel(all_cls_scores, all_line_preds, gt_labels, gt_shifts_pts):
    predT = all_line_preds.reshape(L, B, NP, D).transpose(0, 1, 3, 2)
    gt3 = gt_shifts_pts.reshape(B, NG, S, D).transpose(0, 2, 1, 3)
    gtflat = gt_shifts_pts.reshape(B, NG * S, D)
    clsT = all_cls_scores.transpose(0, 1, 3, 2)
    oh = jax.nn.one_hot(gt_labels, C, dtype=jnp.float32)
    pred_sc = jnp.pad(predT[:, :, :, NTC:].reshape(PROBS, D, NSC),
                      ((0, 0), (0, 0), (0, NSCP - NSC)))
    pred_sc4 = pred_sc.reshape(PROBS, D, 16, 16).transpose(0, 2, 1, 3)

    gtflat48 = jnp.pad(gtflat, ((0, 0), (0, 0), (0, DP - D)))
    scrm = _sc_call(pred_sc4, gtflat48)
    scrm4 = scrm.transpose(0, 2, 1, 3).reshape(L, B, NG, NSCP)

    bgp, cand = pl.pallas_call(
        _tc_cost_kernel,
        grid=(B, L),
        in_specs=[
            pl.BlockSpec((1, 1, D, NP), lambda b, l: (l, b, 0, 0)),
            pl.BlockSpec((1, S, NG, D), lambda b, l: (b, 0, 0, 0)),
            pl.BlockSpec((1, 1, C, NP), lambda b, l: (l, b, 0, 0)),
            pl.BlockSpec((1, NG, C), lambda b, l: (b, 0, 0)),
        ],
        out_specs=[
            pl.BlockSpec((1, 1, 8, 128), lambda b, l: (l, b, 0, 0)),
            pl.BlockSpec((1, 1, NG, 8), lambda b, l: (l, b, 0, 0)),
        ],
        out_shape=[
            jax.ShapeDtypeStruct((L, B, 8, 128), jnp.float32),
            jax.ShapeDtypeStruct((L, B, NG, 8), jnp.float32),
        ],
        scratch_shapes=[pltpu.VMEM((NG, NTC), jnp.float32)],
    )(predT, gt3, clsT, oh)

    parts = pl.pallas_call(
        _merge_kernel,
        grid=(B, L),
        in_specs=[
            pl.BlockSpec((1, 1, NG, NSCP), lambda b, l: (l, b, 0, 0)),
            pl.BlockSpec((1, 1, NG, 8), lambda b, l: (l, b, 0, 0)),
            pl.BlockSpec((1, 1, D, NP), lambda b, l: (l, b, 0, 0)),
            pl.BlockSpec((1, S, NG, D), lambda b, l: (b, 0, 0, 0)),
            pl.BlockSpec((1, NG * S, D), lambda b, l: (b, 0, 0)),
            pl.BlockSpec((1, 1, C, NP), lambda b, l: (l, b, 0, 0)),
            pl.BlockSpec((1, NG, C), lambda b, l: (b, 0, 0)),
        ],
        out_specs=pl.BlockSpec((1, 1, 8, 128), lambda b, l: (l, b, 0, 0)),
        out_shape=jax.ShapeDtypeStruct((L, B, 8, 128), jnp.float32),
    )(scrm4, cand, predT, gt3, gtflat, clsT, oh)

    v = parts[:, :, :, 0]
    bg = bgp[:, :, 3, 0]
    num_pos = jnp.maximum(v[:, :, 0].sum(axis=1), 1.0)
    loss_cls = (bg.sum(axis=1) + v[:, :, 2].sum(axis=1)) / num_pos * W_CLS
    loss_pts = v[:, :, 1].sum(axis=1) / num_pos * W_PTS
    loss_dir = v[:, :, 4].sum(axis=1) / num_pos * W_DIR
    out = jnp.stack([loss_cls, loss_pts, loss_dir], axis=1)
    return jnp.nan_to_num(out)


# speedup vs baseline: 1.0502x; 1.0502x over previous
import functools

import jax
import jax.numpy as jnp
from jax.experimental import pallas as pl
from jax.experimental.pallas import tpu as pltpu

L, B, NP, NG, S, P, C = 6, 4, 1000, 50, 20, 20, 3
D = P * 2
ALPHA, GAMMA = 0.25, 2.0
W_CLS, W_PTS, W_DIR = 2.0, 5.0, 0.005
COST_CLS, COST_PTS = 2.0, 5.0
EPS = 1e-8


def _problem_kernel(predT_r, gt3_r, gtflat_r, clsT_r, oh_r, out_r, rm_ref):
    pred = predT_r[0, 0]
    big = jnp.float32(3.4e38)

    LC = 256

    def s_body(s, _):
        gts = gt3_r[0, s]
        for lo in range(0, NP, LC):
            hi = min(lo + LC, NP)
            acc = jnp.abs(gts[:, 0:1] - pred[0:1, lo:hi])
            for d in range(1, D):
                gcol = gts[:, d:d + 1]
                prow = pred[d:d + 1, lo:hi]
                acc = acc + jnp.abs(gcol - prow)
            rm_ref[:, lo:hi] = jnp.minimum(rm_ref[:, lo:hi], acc)
        return 0

    rm_ref[...] = jnp.full((NG, NP), big, jnp.float32)
    jax.lax.fori_loop(0, S, s_body, 0)
    runmin = rm_ref[...]

    x = clsT_r[0, 0]
    p = jax.nn.sigmoid(x)
    one_m_p = 1.0 - p
    pos = ALPHA * (one_m_p * one_m_p) * (-jnp.log(p + EPS))
    neg = (1.0 - ALPHA) * (p * p) * (-jnp.log(one_m_p + EPS))
    pn = pos - neg
    oh = oh_r[0]
    cls_T = (oh[:, 0:1] * pn[0:1, :]
             + oh[:, 1:2] * pn[1:2, :]
             + oh[:, 2:3] * pn[2:3, :])

    cost = cls_T * COST_CLS + runmin * COST_PTS

    rowmin = jnp.min(cost, axis=1, keepdims=True)
    l_iota = jax.lax.broadcasted_iota(jnp.int32, (NG, NP), 1)
    argidx = jnp.min(jnp.where(cost == rowmin, l_iota, NP),
                     axis=1, keepdims=True)
    onehot_assign = l_iota == argidx

    ng_iota = jax.lax.broadcasted_iota(jnp.int32, (NG, NP), 0)
    colmax = jnp.max(jnp.where(onehot_assign, ng_iota, -1),
                     axis=0, keepdims=True)
    winner = onehot_assign & (ng_iota == colmax)
    wf = winner.astype(jnp.float32)

    num_pos = jnp.sum(wf)
    sum_pts = jnp.sum(wf * runmin)
    sum_corr = jnp.sum(wf * cls_T)
    bg = jnp.sum(neg)

    roww = jnp.max(wf, axis=1, keepdims=True)
    gpred = jax.lax.dot_general(
        wf, pred, (((1,), (1,)), ((), ())),
        preferred_element_type=jnp.float32)

    srunmin = jnp.full((NG, 1), big, jnp.float32)
    srunarg = jnp.zeros((NG, 1), jnp.int32)
    for s in range(S):
        g = gt3_r[0, s]
        dsum = jnp.sum(jnp.abs(gpred - g), axis=1, keepdims=True)
        better = dsum < srunmin
        srunarg = jnp.where(better, s, srunarg)
        srunmin = jnp.minimum(srunmin, dsum)

    ng_col = jax.lax.broadcasted_iota(jnp.int32, (NG, 1), 0)
    jcol = ng_col * S + srunarg
    j_iota = jax.lax.broadcasted_iota(jnp.int32, (NG, NG * S), 1)
    ohns = (j_iota == jcol).astype(jnp.float32)
    gtd = jax.lax.dot_general(
        ohns, gtflat_r[0], (((1,), (0,)), ((), ())),
        preferred_element_type=jnp.float32)

    pd = gpred[:, 2:] - gpred[:, :-2]
    td = gtd[:, 2:] - gtd[:, :-2]
    r2 = jax.lax.broadcasted_iota(jnp.int32, (D - 2, P - 1), 0) // 2
    c2 = jax.lax.broadcasted_iota(jnp.int32, (D - 2, P - 1), 1)
    pair_m = (r2 == c2).astype(jnp.float32)
    stk = jnp.concatenate([pd * td, pd * pd, td * td], axis=0)
    res = jax.lax.dot_general(stk, pair_m, (((1,), (0,)), ((), ())),
                              preferred_element_type=jnp.float32)
    dots = res[0:NG]
    pdn = res[NG:2 * NG]
    tdn = res[2 * NG:3 * NG]
    cos = dots / (jnp.sqrt(pdn) * jnp.sqrt(tdn) + EPS)
    sum_dir = jnp.sum((1.0 - cos) * roww)

    riota = jax.lax.broadcasted_iota(jnp.int32, (8, 128), 0)
    arr = (jnp.where(riota == 0, num_pos, 0.0)
           + jnp.where(riota == 1, sum_pts, 0.0)
           + jnp.where(riota == 2, sum_corr, 0.0)
           + jnp.where(riota == 3, bg, 0.0)
           + jnp.where(riota == 4, sum_dir, 0.0))
    out_r[0, 0] = arr


@jax.jit
def kernel(all_cls_scores, all_line_preds, gt_labels, gt_shifts_pts):
    predn = all_line_preds.reshape(L, B, NP, D)
    predT = predn.transpose(0, 1, 3, 2)
    gt3 = gt_shifts_pts.reshape(B, NG, S, D).transpose(0, 2, 1, 3)
    gtflat = gt_shifts_pts.reshape(B, NG * S, D)
    clsT = all_cls_scores.transpose(0, 1, 3, 2)
    oh = jax.nn.one_hot(gt_labels, C, dtype=jnp.float32)

    parts = pl.pallas_call(
        _problem_kernel,
        grid=(B, L),
        in_specs=[
            pl.BlockSpec((1, 1, D, NP), lambda b, l: (l, b, 0, 0)),
            pl.BlockSpec((1, S, NG, D), lambda b, l: (b, 0, 0, 0)),
            pl.BlockSpec((1, NG * S, D), lambda b, l: (b, 0, 0)),
            pl.BlockSpec((1, 1, C, NP), lambda b, l: (l, b, 0, 0)),
            pl.BlockSpec((1, NG, C), lambda b, l: (b, 0, 0)),
        ],
        out_specs=pl.BlockSpec((1, 1, 8, 128), lambda b, l: (l, b, 0, 0)),
        out_shape=jax.ShapeDtypeStruct((L, B, 8, 128), jnp.float32),
        scratch_shapes=[pltpu.VMEM((NG, NP), jnp.float32)],
    )(predT, gt3, gtflat, clsT, oh)

    v = parts[:, :, :, 0]
    num_pos = jnp.maximum(v[:, :, 0].sum(axis=1), 1.0)
    loss_cls = (v[:, :, 3].sum(axis=1) + v[:, :, 2].sum(axis=1)) / num_pos * W_CLS
    loss_pts = v[:, :, 1].sum(axis=1) / num_pos * W_PTS
    loss_dir = v[:, :, 4].sum(axis=1) / num_pos * W_DIR
    out = jnp.stack([loss_cls, loss_pts, loss_dir], axis=1)
    return jnp.nan_to_num(out)
